# Initial kernel scaffold; baseline (speedup 1.0000x reference)
import jax
import jax.numpy as jnp
from jax.experimental import pallas as pl


def kernel(x, edge_index, batch, emb_tables, W_pre, b_pre, W_post, b_post, W_lin, b_lin, bn_gamma, bn_beta, W_mlp, b_mlp):
    raise NotImplementedError("write your pallas kernel here")


# speedup vs baseline: 1.2379x; 1.2379x over previous
import functools
import jax
import jax.numpy as jnp
import numpy as np
from jax.experimental import pallas as pl

HID = 128
NUM_LAYERS = 4
N_NODES = 10000
N_GRAPHS = 512
_DEG_HIST = np.array([0,0,0,0,0,0,0,0,0,0,0,0,0,0,0,0,120,260,420,560,680,760,800,820,800,760,680,560,420,260,120,60,30,15,8,4,2,1], dtype=np.float64)
_DELTA = float((_DEG_HIST * np.log(np.arange(len(_DEG_HIST)) + 1.0)).sum() / _DEG_HIST.sum())

_TILE = 1000
_GRID = N_NODES // _TILE


def _pre_body(h_ref, w_ref, b_ref, a_ref, u_ref, u2_ref):
    h = h_ref[...]
    wd = w_ref[0:HID, :]
    ws = w_ref[HID:2 * HID, :]
    a_ref[...] = jnp.dot(h, wd, preferred_element_type=jnp.float32) + b_ref[...]
    u = jnp.dot(h, ws, preferred_element_type=jnp.float32)
    u_ref[...] = u
    u2_ref[...] = u * u


def _pre_call(h, w_pre, b_pre):
    return pl.pallas_call(
        _pre_body,
        grid=(_GRID,),
        in_specs=[
            pl.BlockSpec((_TILE, HID), lambda i: (i, 0)),
            pl.BlockSpec((2 * HID, HID), lambda i: (0, 0)),
            pl.BlockSpec((1, HID), lambda i: (0, 0)),
        ],
        out_specs=[
            pl.BlockSpec((_TILE, HID), lambda i: (i, 0)),
            pl.BlockSpec((_TILE, HID), lambda i: (i, 0)),
            pl.BlockSpec((_TILE, HID), lambda i: (i, 0)),
        ],
        out_shape=[jax.ShapeDtypeStruct((N_NODES, HID), jnp.float32)] * 3,
    )(h, w_pre, b_pre)


def _post_body(h_ref, a_ref, s1_ref, s2_ref, mn_ref, mx_ref, deg_ref,
               wpost_ref, bpost_ref, wlin_ref, blin_ref, t_ref, stats_ref):
    h = h_ref[...]
    a = a_ref[...]
    s1 = s1_ref[...]
    s2 = s2_ref[...]
    deg = deg_ref[...]
    degc = jnp.maximum(deg, 1.0)
    has = deg > 0.0

    mean = (deg * a + s1) / degc
    msq = (deg * a * a + 2.0 * a * s1 + s2) / degc
    std = jnp.sqrt(jax.nn.relu(msq - mean * mean) + 1e-5)
    mn = jnp.where(has, a + mn_ref[...], 0.0)
    mx = jnp.where(has, a + mx_ref[...], 0.0)
    agg = jnp.concatenate([mean, mn, mx, std], axis=-1)

    logd = jnp.log(deg + 1.0)
    amp = logd / _DELTA
    att = _DELTA / jnp.maximum(logd, 1e-6)

    w0 = wpost_ref[0:HID, :]
    w1 = wpost_ref[HID:5 * HID, :]
    w2 = wpost_ref[5 * HID:9 * HID, :]
    w3 = wpost_ref[9 * HID:13 * HID, :]
    t = (jnp.dot(h, w0, preferred_element_type=jnp.float32)
         + jnp.dot(agg, w1, preferred_element_type=jnp.float32)
         + amp * jnp.dot(agg, w2, preferred_element_type=jnp.float32)
         + att * jnp.dot(agg, w3, preferred_element_type=jnp.float32)
         + bpost_ref[...])
    t = jnp.dot(t, wlin_ref[...], preferred_element_type=jnp.float32) + blin_ref[...]
    t_ref[...] = t

    part = jnp.concatenate(
        [jnp.sum(t, axis=0, keepdims=True),
         jnp.sum(t * t, axis=0, keepdims=True),
         jnp.zeros((6, HID), jnp.float32)], axis=0)

    @pl.when(pl.program_id(0) == 0)
    def _init():
        stats_ref[...] = jnp.zeros_like(stats_ref)

    stats_ref[...] += part


def _post_call(h, a, s1, s2, mn, mx, deg2d, w_post, b_post, w_lin, b_lin):
    tile_spec = pl.BlockSpec((_TILE, HID), lambda i: (i, 0))
    return pl.pallas_call(
        _post_body,
        grid=(_GRID,),
        in_specs=[
            tile_spec, tile_spec, tile_spec, tile_spec, tile_spec, tile_spec,
            pl.BlockSpec((_TILE, 1), lambda i: (i, 0)),
            pl.BlockSpec((13 * HID, HID), lambda i: (0, 0)),
            pl.BlockSpec((1, HID), lambda i: (0, 0)),
            pl.BlockSpec((HID, HID), lambda i: (0, 0)),
            pl.BlockSpec((1, HID), lambda i: (0, 0)),
        ],
        out_specs=[
            tile_spec,
            pl.BlockSpec((8, HID), lambda i: (0, 0)),
        ],
        out_shape=[
            jax.ShapeDtypeStruct((N_NODES, HID), jnp.float32),
            jax.ShapeDtypeStruct((8, HID), jnp.float32),
        ],
    )(h, a, s1, s2, mn, mx, deg2d, w_post, b_post, w_lin, b_lin)


def _bnres_body(t_ref, h_ref, stats_ref, g_ref, b_ref, out_ref):
    t = t_ref[...]
    mu = stats_ref[0:1, :] / float(N_NODES)
    var = stats_ref[1:2, :] / float(N_NODES) - mu * mu
    y = (t - mu) / jnp.sqrt(var + 1e-5) * g_ref[...] + b_ref[...]
    out_ref[...] = h_ref[...] + jax.nn.relu(y)


def _bnres_call(t, h, stats, gamma, beta):
    tile_spec = pl.BlockSpec((_TILE, HID), lambda i: (i, 0))
    return pl.pallas_call(
        _bnres_body,
        grid=(_GRID,),
        in_specs=[
            tile_spec, tile_spec,
            pl.BlockSpec((8, HID), lambda i: (0, 0)),
            pl.BlockSpec((1, HID), lambda i: (0, 0)),
            pl.BlockSpec((1, HID), lambda i: (0, 0)),
        ],
        out_specs=tile_spec,
        out_shape=jax.ShapeDtypeStruct((N_NODES, HID), jnp.float32),
    )(t, h, stats, gamma, beta)


def _pool_body(h_ref, batch_ref, wmlp_ref, bmlp_ref, out_ref):
    b = batch_ref[...]
    gi = jax.lax.broadcasted_iota(jnp.int32, (N_GRAPHS, N_NODES), 0)
    oh = (b == gi).astype(jnp.float32)
    cnt = jnp.sum(oh, axis=1, keepdims=True)
    pooled = jnp.dot(oh, h_ref[...], preferred_element_type=jnp.float32)
    pooled = pooled / jnp.maximum(cnt, 1.0)
    out_ref[...] = jnp.dot(pooled, wmlp_ref[...], preferred_element_type=jnp.float32) + bmlp_ref[...]


def _pool_call(h, batch2d, w_mlp, b_mlp):
    out_dim = w_mlp.shape[1]
    return pl.pallas_call(
        _pool_body,
        in_specs=[
            pl.BlockSpec((N_NODES, HID), lambda: (0, 0)),
            pl.BlockSpec((1, N_NODES), lambda: (0, 0)),
            pl.BlockSpec((HID, out_dim), lambda: (0, 0)),
            pl.BlockSpec((1, out_dim), lambda: (0, 0)),
        ],
        out_specs=pl.BlockSpec((N_GRAPHS, out_dim), lambda: (0, 0)),
        out_shape=jax.ShapeDtypeStruct((N_GRAPHS, out_dim), jnp.float32),
    )(h, batch2d, w_mlp, b_mlp)


@jax.jit
def kernel(x, edge_index, batch, emb_tables, W_pre, b_pre, W_post, b_post,
           W_lin, b_lin, bn_gamma, bn_beta, W_mlp, b_mlp):
    h = emb_tables[0][x[:, 0]]
    for i in range(1, 9):
        h = h + emb_tables[i][x[:, i]]

    src, dst = edge_index[0], edge_index[1]
    deg = jax.ops.segment_sum(jnp.ones((src.shape[0],), jnp.float32), dst, N_NODES)
    deg2d = deg[:, None]

    for l in range(NUM_LAYERS):
        a, u, u2 = _pre_call(h, W_pre[l], b_pre[l][None, :])
        us = u[src]
        s1 = jax.ops.segment_sum(us, dst, N_NODES)
        s2 = jax.ops.segment_sum(u2[src], dst, N_NODES)
        mn = jax.ops.segment_min(us, dst, N_NODES)
        mx = jax.ops.segment_max(us, dst, N_NODES)
        t, stats = _post_call(h, a, s1, s2, mn, mx, deg2d, W_post[l],
                              b_post[l][None, :], W_lin[l], b_lin[l][None, :])
        h = _bnres_call(t, h, stats, bn_gamma[l][None, :], bn_beta[l][None, :])

    return _pool_call(h, batch.astype(jnp.int32)[None, :], W_mlp, b_mlp[None, :])


# speedup vs baseline: 1.3036x; 1.0531x over previous
import functools
import jax
import jax.numpy as jnp
import numpy as np
from jax.experimental import pallas as pl

HID = 128
NUM_LAYERS = 4
N_NODES = 10000
N_GRAPHS = 512
_DEG_HIST = np.array([0,0,0,0,0,0,0,0,0,0,0,0,0,0,0,0,120,260,420,560,680,760,800,820,800,760,680,560,420,260,120,60,30,15,8,4,2,1], dtype=np.float64)
_DELTA = float((_DEG_HIST * np.log(np.arange(len(_DEG_HIST)) + 1.0)).sum() / _DEG_HIST.sum())

_TILE = 1000
_GRID = N_NODES // _TILE


def _pre_body(h_ref, w_ref, b_ref, a_ref, u_ref):
    h = h_ref[...]
    wd = w_ref[0:HID, :]
    ws = w_ref[HID:2 * HID, :]
    a_ref[...] = jnp.dot(h, wd, preferred_element_type=jnp.float32) + b_ref[...]
    u_ref[...] = jnp.dot(h, ws, preferred_element_type=jnp.float32)


def _pre_call(h, w_pre, b_pre):
    return pl.pallas_call(
        _pre_body,
        grid=(_GRID,),
        in_specs=[
            pl.BlockSpec((_TILE, HID), lambda i: (i, 0)),
            pl.BlockSpec((2 * HID, HID), lambda i: (0, 0)),
            pl.BlockSpec((1, HID), lambda i: (0, 0)),
        ],
        out_specs=[
            pl.BlockSpec((_TILE, HID), lambda i: (i, 0)),
            pl.BlockSpec((_TILE, HID), lambda i: (i, 0)),
        ],
        out_shape=[jax.ShapeDtypeStruct((N_NODES, HID), jnp.float32)] * 2,
    )(h, w_pre, b_pre)


def _post_body(h_ref, a_ref, s1_ref, s2_ref, mn_ref, mx_ref, deg_ref,
               wpost_ref, bpost_ref, wlin_ref, blin_ref, t_ref, stats_ref):
    h = h_ref[...]
    a = a_ref[...]
    s1 = s1_ref[...]
    s2 = s2_ref[...]
    deg = deg_ref[...]
    degc = jnp.maximum(deg, 1.0)
    has = deg > 0.0

    mean = (deg * a + s1) / degc
    msq = (deg * a * a + 2.0 * a * s1 + s2) / degc
    std = jnp.sqrt(jax.nn.relu(msq - mean * mean) + 1e-5)
    mn = jnp.where(has, a + mn_ref[...], 0.0)
    mx = jnp.where(has, a + mx_ref[...], 0.0)
    agg = jnp.concatenate([mean, mn, mx, std], axis=-1)

    logd = jnp.log(deg + 1.0)
    amp = logd / _DELTA
    att = _DELTA / jnp.maximum(logd, 1e-6)

    w0 = wpost_ref[0:HID, :]
    w1 = wpost_ref[HID:5 * HID, :]
    w2 = wpost_ref[5 * HID:9 * HID, :]
    w3 = wpost_ref[9 * HID:13 * HID, :]
    t = (jnp.dot(h, w0, preferred_element_type=jnp.float32)
         + jnp.dot(agg, w1, preferred_element_type=jnp.float32)
         + amp * jnp.dot(agg, w2, preferred_element_type=jnp.float32)
         + att * jnp.dot(agg, w3, preferred_element_type=jnp.float32)
         + bpost_ref[...])
    t = jnp.dot(t, wlin_ref[...], preferred_element_type=jnp.float32) + blin_ref[...]
    t_ref[...] = t

    part = jnp.concatenate(
        [jnp.sum(t, axis=0, keepdims=True),
         jnp.sum(t * t, axis=0, keepdims=True),
         jnp.zeros((6, HID), jnp.float32)], axis=0)

    @pl.when(pl.program_id(0) == 0)
    def _init():
        stats_ref[...] = jnp.zeros_like(stats_ref)

    stats_ref[...] += part


def _post_call(h, a, s1, s2, mn, mx, deg2d, w_post, b_post, w_lin, b_lin):
    tile_spec = pl.BlockSpec((_TILE, HID), lambda i: (i, 0))
    return pl.pallas_call(
        _post_body,
        grid=(_GRID,),
        in_specs=[
            tile_spec, tile_spec, tile_spec, tile_spec, tile_spec, tile_spec,
            pl.BlockSpec((_TILE, 1), lambda i: (i, 0)),
            pl.BlockSpec((13 * HID, HID), lambda i: (0, 0)),
            pl.BlockSpec((1, HID), lambda i: (0, 0)),
            pl.BlockSpec((HID, HID), lambda i: (0, 0)),
            pl.BlockSpec((1, HID), lambda i: (0, 0)),
        ],
        out_specs=[
            tile_spec,
            pl.BlockSpec((8, HID), lambda i: (0, 0)),
        ],
        out_shape=[
            jax.ShapeDtypeStruct((N_NODES, HID), jnp.float32),
            jax.ShapeDtypeStruct((8, HID), jnp.float32),
        ],
    )(h, a, s1, s2, mn, mx, deg2d, w_post, b_post, w_lin, b_lin)


def _bnres_body(t_ref, h_ref, stats_ref, g_ref, b_ref, out_ref):
    t = t_ref[...]
    mu = stats_ref[0:1, :] / float(N_NODES)
    var = stats_ref[1:2, :] / float(N_NODES) - mu * mu
    y = (t - mu) / jnp.sqrt(var + 1e-5) * g_ref[...] + b_ref[...]
    out_ref[...] = h_ref[...] + jax.nn.relu(y)


def _bnres_call(t, h, stats, gamma, beta):
    tile_spec = pl.BlockSpec((_TILE, HID), lambda i: (i, 0))
    return pl.pallas_call(
        _bnres_body,
        grid=(_GRID,),
        in_specs=[
            tile_spec, tile_spec,
            pl.BlockSpec((8, HID), lambda i: (0, 0)),
            pl.BlockSpec((1, HID), lambda i: (0, 0)),
            pl.BlockSpec((1, HID), lambda i: (0, 0)),
        ],
        out_specs=tile_spec,
        out_shape=jax.ShapeDtypeStruct((N_NODES, HID), jnp.float32),
    )(t, h, stats, gamma, beta)


def _pool_body(h_ref, batch_ref, wmlp_ref, bmlp_ref, out_ref):
    b = batch_ref[...]
    gi = jax.lax.broadcasted_iota(jnp.int32, (N_GRAPHS, N_NODES), 0)
    oh = (b == gi).astype(jnp.float32)
    cnt = jnp.sum(oh, axis=1, keepdims=True)
    pooled = jnp.dot(oh, h_ref[...], preferred_element_type=jnp.float32)
    pooled = pooled / jnp.maximum(cnt, 1.0)
    out_ref[...] = jnp.dot(pooled, wmlp_ref[...], preferred_element_type=jnp.float32) + bmlp_ref[...]


def _pool_call(h, batch2d, w_mlp, b_mlp):
    out_dim = w_mlp.shape[1]
    return pl.pallas_call(
        _pool_body,
        in_specs=[
            pl.BlockSpec((N_NODES, HID), lambda: (0, 0)),
            pl.BlockSpec((1, N_NODES), lambda: (0, 0)),
            pl.BlockSpec((HID, out_dim), lambda: (0, 0)),
            pl.BlockSpec((1, out_dim), lambda: (0, 0)),
        ],
        out_specs=pl.BlockSpec((N_GRAPHS, out_dim), lambda: (0, 0)),
        out_shape=jax.ShapeDtypeStruct((N_GRAPHS, out_dim), jnp.float32),
    )(h, batch2d, w_mlp, b_mlp)


@jax.jit
def kernel(x, edge_index, batch, emb_tables, W_pre, b_pre, W_post, b_post,
           W_lin, b_lin, bn_gamma, bn_beta, W_mlp, b_mlp):
    h = emb_tables[0][x[:, 0]]
    for i in range(1, 9):
        h = h + emb_tables[i][x[:, i]]

    order = jnp.argsort(edge_index[1])
    src = edge_index[0][order]
    dst = edge_index[1][order]
    deg = jax.ops.segment_sum(jnp.ones((src.shape[0],), jnp.float32), dst,
                              N_NODES, indices_are_sorted=True)
    deg2d = deg[:, None]

    for l in range(NUM_LAYERS):
        a, u = _pre_call(h, W_pre[l], b_pre[l][None, :])
        us = u[src]
        s12 = jax.ops.segment_sum(jnp.concatenate([us, us * us], axis=1), dst,
                                  N_NODES, indices_are_sorted=True)
        s1, s2 = s12[:, :HID], s12[:, HID:]
        mnmx = jax.ops.segment_min(jnp.concatenate([us, -us], axis=1), dst,
                                   N_NODES, indices_are_sorted=True)
        mn, mx = mnmx[:, :HID], -mnmx[:, HID:]
        t, stats = _post_call(h, a, s1, s2, mn, mx, deg2d, W_post[l],
                              b_post[l][None, :], W_lin[l], b_lin[l][None, :])
        h = _bnres_call(t, h, stats, bn_gamma[l][None, :], bn_beta[l][None, :])

    return _pool_call(h, batch.astype(jnp.int32)[None, :], W_mlp, b_mlp[None, :])


# speedup vs baseline: 1.8642x; 1.4301x over previous
import functools
import jax
import jax.numpy as jnp
import numpy as np
from jax.experimental import pallas as pl
from jax.experimental.pallas import tpu as pltpu

HID = 128
NUM_LAYERS = 4
N_NODES = 10000
N_GRAPHS = 512
_DEG_HIST = np.array([0,0,0,0,0,0,0,0,0,0,0,0,0,0,0,0,120,260,420,560,680,760,800,820,800,760,680,560,420,260,120,60,30,15,8,4,2,1], dtype=np.float64)
_DELTA = float((_DEG_HIST * np.log(np.arange(len(_DEG_HIST)) + 1.0)).sum() / _DEG_HIST.sum())

_TILE = 1000
_GRID = N_NODES // _TILE


def _pre_body(h_ref, w_ref, b_ref, a_ref, u_ref):
    h = h_ref[...]
    wd = w_ref[0:HID, :]
    ws = w_ref[HID:2 * HID, :]
    a_ref[...] = jnp.dot(h, wd, preferred_element_type=jnp.float32) + b_ref[...]
    u_ref[...] = jnp.dot(h, ws, preferred_element_type=jnp.float32)


def _pre_call(h, w_pre, b_pre):
    return pl.pallas_call(
        _pre_body,
        grid=(_GRID,),
        in_specs=[
            pl.BlockSpec((_TILE, HID), lambda i: (i, 0)),
            pl.BlockSpec((2 * HID, HID), lambda i: (0, 0)),
            pl.BlockSpec((1, HID), lambda i: (0, 0)),
        ],
        out_specs=[
            pl.BlockSpec((_TILE, HID), lambda i: (i, 0)),
            pl.BlockSpec((_TILE, HID), lambda i: (i, 0)),
        ],
        out_shape=[jax.ShapeDtypeStruct((N_NODES, HID), jnp.float32)] * 2,
    )(h, w_pre, b_pre)


_EC = 256
_NB = 320000 // _EC
_BIG = 3.0e38


def _seg_body(us_ref, lr_ref, f_ref, uniq_ref, d_ref, out1_ref, out2_ref,
              psum_ref, pmin_ref):
    @pl.when(pl.program_id(0) == 0)
    def _init():
        out1_ref[...] = jnp.zeros_like(out1_ref)
        out2_ref[...] = jnp.full_like(out2_ref, _BIG)

    us = us_ref[...]
    lr = lr_ref[...]
    f = f_ref[...]
    cat = jnp.concatenate([us, us * us], axis=1)
    m = jnp.concatenate([us, -us], axis=1)

    iota_r = jax.lax.broadcasted_iota(jnp.int32, (_EC, _EC), 1)
    oh = (lr == iota_r).astype(jnp.float32)
    psum_ref[...] = jax.lax.dot_general(
        oh, cat, (((0,), (0,)), ((), ())), preferred_element_type=jnp.float32)

    s = 1
    while s < _EC:
        m_s = jnp.concatenate([m[s:], jnp.full((s, 2 * HID), _BIG)], axis=0)
        lr_s = jnp.concatenate([lr[s:], jnp.full((s, 1), -1, jnp.int32)], axis=0)
        m = jnp.minimum(m, jnp.where(lr_s == lr, m_s, _BIG))
        s *= 2
    sel = oh * f
    pmin_ref[...] = jax.lax.dot_general(
        sel, jnp.where(f > 0, m, 0.0), (((0,), (0,)), ((), ())),
        preferred_element_type=jnp.float32)

    def body(j, carry):
        r = uniq_ref[j]
        out1_ref[pl.ds(r, 1), :] = out1_ref[pl.ds(r, 1), :] + psum_ref[pl.ds(j, 1), :]
        out2_ref[pl.ds(r, 1), :] = jnp.minimum(out2_ref[pl.ds(r, 1), :],
                                               pmin_ref[pl.ds(j, 1), :])
        return carry

    jax.lax.fori_loop(0, d_ref[0], body, 0)


def _seg_call(us, lr, f, uniq, d):
    return pl.pallas_call(
        _seg_body,
        grid=(_NB,),
        in_specs=[
            pl.BlockSpec((_EC, HID), lambda i: (i, 0)),
            pl.BlockSpec((_EC, 1), lambda i: (i, 0)),
            pl.BlockSpec((_EC, 1), lambda i: (i, 0)),
            pl.BlockSpec((_EC,), lambda i: (i,), memory_space=pltpu.SMEM),
            pl.BlockSpec((128,), lambda i: (i,), memory_space=pltpu.SMEM),
        ],
        out_specs=[
            pl.BlockSpec((N_NODES, 2 * HID), lambda i: (0, 0)),
            pl.BlockSpec((N_NODES, 2 * HID), lambda i: (0, 0)),
        ],
        out_shape=[jax.ShapeDtypeStruct((N_NODES, 2 * HID), jnp.float32)] * 2,
        scratch_shapes=[
            pltpu.VMEM((_EC, 2 * HID), jnp.float32),
            pltpu.VMEM((_EC, 2 * HID), jnp.float32),
        ],
    )(us, lr, f, uniq, d)


def _post_body(h_ref, a_ref, s1_ref, s2_ref, mn_ref, mx_ref, deg_ref,
               wpost_ref, bpost_ref, wlin_ref, blin_ref, t_ref, stats_ref):
    h = h_ref[...]
    a = a_ref[...]
    s1 = s1_ref[...]
    s2 = s2_ref[...]
    deg = deg_ref[...]
    degc = jnp.maximum(deg, 1.0)
    has = deg > 0.0

    mean = (deg * a + s1) / degc
    msq = (deg * a * a + 2.0 * a * s1 + s2) / degc
    std = jnp.sqrt(jax.nn.relu(msq - mean * mean) + 1e-5)
    mn = jnp.where(has, a + mn_ref[...], 0.0)
    mx = jnp.where(has, a + mx_ref[...], 0.0)
    agg = jnp.concatenate([mean, mn, mx, std], axis=-1)

    logd = jnp.log(deg + 1.0)
    amp = logd / _DELTA
    att = _DELTA / jnp.maximum(logd, 1e-6)

    w0 = wpost_ref[0:HID, :]
    w1 = wpost_ref[HID:5 * HID, :]
    w2 = wpost_ref[5 * HID:9 * HID, :]
    w3 = wpost_ref[9 * HID:13 * HID, :]
    t = (jnp.dot(h, w0, preferred_element_type=jnp.float32)
         + jnp.dot(agg, w1, preferred_element_type=jnp.float32)
         + amp * jnp.dot(agg, w2, preferred_element_type=jnp.float32)
         + att * jnp.dot(agg, w3, preferred_element_type=jnp.float32)
         + bpost_ref[...])
    t = jnp.dot(t, wlin_ref[...], preferred_element_type=jnp.float32) + blin_ref[...]
    t_ref[...] = t

    part = jnp.concatenate(
        [jnp.sum(t, axis=0, keepdims=True),
         jnp.sum(t * t, axis=0, keepdims=True),
         jnp.zeros((6, HID), jnp.float32)], axis=0)

    @pl.when(pl.program_id(0) == 0)
    def _init():
        stats_ref[...] = jnp.zeros_like(stats_ref)

    stats_ref[...] += part


def _post_call(h, a, s1, s2, mn, mx, deg2d, w_post, b_post, w_lin, b_lin):
    tile_spec = pl.BlockSpec((_TILE, HID), lambda i: (i, 0))
    return pl.pallas_call(
        _post_body,
        grid=(_GRID,),
        in_specs=[
            tile_spec, tile_spec, tile_spec, tile_spec, tile_spec, tile_spec,
            pl.BlockSpec((_TILE, 1), lambda i: (i, 0)),
            pl.BlockSpec((13 * HID, HID), lambda i: (0, 0)),
            pl.BlockSpec((1, HID), lambda i: (0, 0)),
            pl.BlockSpec((HID, HID), lambda i: (0, 0)),
            pl.BlockSpec((1, HID), lambda i: (0, 0)),
        ],
        out_specs=[
            tile_spec,
            pl.BlockSpec((8, HID), lambda i: (0, 0)),
        ],
        out_shape=[
            jax.ShapeDtypeStruct((N_NODES, HID), jnp.float32),
            jax.ShapeDtypeStruct((8, HID), jnp.float32),
        ],
    )(h, a, s1, s2, mn, mx, deg2d, w_post, b_post, w_lin, b_lin)


def _bnres_body(t_ref, h_ref, stats_ref, g_ref, b_ref, out_ref):
    t = t_ref[...]
    mu = stats_ref[0:1, :] / float(N_NODES)
    var = stats_ref[1:2, :] / float(N_NODES) - mu * mu
    y = (t - mu) / jnp.sqrt(var + 1e-5) * g_ref[...] + b_ref[...]
    out_ref[...] = h_ref[...] + jax.nn.relu(y)


def _bnres_call(t, h, stats, gamma, beta):
    tile_spec = pl.BlockSpec((_TILE, HID), lambda i: (i, 0))
    return pl.pallas_call(
        _bnres_body,
        grid=(_GRID,),
        in_specs=[
            tile_spec, tile_spec,
            pl.BlockSpec((8, HID), lambda i: (0, 0)),
            pl.BlockSpec((1, HID), lambda i: (0, 0)),
            pl.BlockSpec((1, HID), lambda i: (0, 0)),
        ],
        out_specs=tile_spec,
        out_shape=jax.ShapeDtypeStruct((N_NODES, HID), jnp.float32),
    )(t, h, stats, gamma, beta)


def _pool_body(h_ref, batch_ref, wmlp_ref, bmlp_ref, out_ref):
    b = batch_ref[...]
    gi = jax.lax.broadcasted_iota(jnp.int32, (N_GRAPHS, N_NODES), 0)
    oh = (b == gi).astype(jnp.float32)
    cnt = jnp.sum(oh, axis=1, keepdims=True)
    pooled = jnp.dot(oh, h_ref[...], preferred_element_type=jnp.float32)
    pooled = pooled / jnp.maximum(cnt, 1.0)
    out_ref[...] = jnp.dot(pooled, wmlp_ref[...], preferred_element_type=jnp.float32) + bmlp_ref[...]


def _pool_call(h, batch2d, w_mlp, b_mlp):
    out_dim = w_mlp.shape[1]
    return pl.pallas_call(
        _pool_body,
        in_specs=[
            pl.BlockSpec((N_NODES, HID), lambda: (0, 0)),
            pl.BlockSpec((1, N_NODES), lambda: (0, 0)),
            pl.BlockSpec((HID, out_dim), lambda: (0, 0)),
            pl.BlockSpec((1, out_dim), lambda: (0, 0)),
        ],
        out_specs=pl.BlockSpec((N_GRAPHS, out_dim), lambda: (0, 0)),
        out_shape=jax.ShapeDtypeStruct((N_GRAPHS, out_dim), jnp.float32),
    )(h, batch2d, w_mlp, b_mlp)


@jax.jit
def kernel(x, edge_index, batch, emb_tables, W_pre, b_pre, W_post, b_post,
           W_lin, b_lin, bn_gamma, bn_beta, W_mlp, b_mlp):
    h = emb_tables[0][x[:, 0]]
    for i in range(1, 9):
        h = h + emb_tables[i][x[:, i]]

    order = jnp.argsort(edge_index[1])
    src = edge_index[0][order]
    dst = edge_index[1][order]
    deg = jax.ops.segment_sum(jnp.ones((src.shape[0],), jnp.float32), dst,
                              N_NODES, indices_are_sorted=True)
    deg2d = deg[:, None]

    n_e = src.shape[0]
    pos = jnp.arange(n_e, dtype=jnp.int32)
    fb = jnp.concatenate([jnp.ones((1,), jnp.bool_), dst[1:] != dst[:-1]])
    fb = fb | (pos % _EC == 0)
    rk = jnp.cumsum(fb.astype(jnp.int32)) - 1
    lr = rk - jnp.repeat(rk[::_EC], _EC)
    d_blk = lr.reshape(_NB, _EC)[:, -1] + 1
    uniq = jnp.zeros((_NB, _EC), jnp.int32).at[pos // _EC, lr].set(dst).reshape(-1)
    d_rep = jnp.repeat(d_blk, 128)
    lr2d = lr[:, None]
    f2d = fb.astype(jnp.float32)[:, None]

    for l in range(NUM_LAYERS):
        a, u = _pre_call(h, W_pre[l], b_pre[l][None, :])
        us = u[src]
        out1, out2 = _seg_call(us, lr2d, f2d, uniq, d_rep)
        s1, s2 = out1[:, :HID], out1[:, HID:]
        mn, mx = out2[:, :HID], -out2[:, HID:]
        t, stats = _post_call(h, a, s1, s2, mn, mx, deg2d, W_post[l],
                              b_post[l][None, :], W_lin[l], b_lin[l][None, :])
        h = _bnres_call(t, h, stats, bn_gamma[l][None, :], bn_beta[l][None, :])

    return _pool_call(h, batch.astype(jnp.int32)[None, :], W_mlp, b_mlp[None, :])
